# baseline (speedup 1.0000x reference)
import functools

import jax
import jax.numpy as jnp
from jax import lax
from jax.experimental import pallas as pl
from jax.experimental.pallas import tpu as pltpu
from jax.experimental.pallas import tpu_sc as plsc

_NC = 2
_NS = 16
_NW = _NC * _NS
_L = 16
_F = 32
_CH = 128


@functools.lru_cache(maxsize=None)
def _build(batch: int):
  assert batch % (_NW * _CH) == 0
  b_per_w = batch // _NW
  n_chunks = b_per_w // _CH
  n_groups = b_per_w // _L
  mesh = plsc.VectorSubcoreMesh(core_axis_name="c", subcore_axis_name="s")

  @functools.partial(
      pl.kernel,
      mesh=mesh,
      out_type=jax.ShapeDtypeStruct((batch,), jnp.float32),
      compiler_params=pltpu.CompilerParams(
          needs_layout_passes=False, use_tc_tiling_on_sc=False),
      scratch_types=[
          pltpu.VMEM((n_chunks, _CH), jnp.int32),
          pltpu.VMEM((n_chunks, _CH), jnp.int32),
          pltpu.VMEM((b_per_w, _F), jnp.float32),
          pltpu.VMEM((b_per_w, _F), jnp.float32),
          pltpu.VMEM((b_per_w,), jnp.float32),
          pltpu.VMEM((b_per_w,), jnp.float32),
          pltpu.VMEM((b_per_w,), jnp.float32),
          pltpu.SemaphoreType.DMA,
      ],
  )
  def mf_kernel(user_hbm, item_hbm, uf_hbm, if_hbm, ub_hbm, ib_hbm,
                out_hbm, idx_u, idx_i, rows_u, rows_i, b_u, b_i, out_v,
                sem):
    wid = lax.axis_index("s") * _NC + lax.axis_index("c")
    base = wid * b_per_w

    copies = []
    for j in range(n_chunks):
      pltpu.sync_copy(user_hbm.at[pl.ds(base + j * _CH, _CH)], idx_u.at[j])
      pltpu.sync_copy(item_hbm.at[pl.ds(base + j * _CH, _CH)], idx_i.at[j])
      copies.append(pltpu.async_copy(
          uf_hbm.at[idx_u.at[j]], rows_u.at[pl.ds(j * _CH, _CH), :], sem))
      copies.append(pltpu.async_copy(
          if_hbm.at[idx_i.at[j]], rows_i.at[pl.ds(j * _CH, _CH), :], sem))
      copies.append(pltpu.async_copy(
          ub_hbm.at[idx_u.at[j]], b_u.at[pl.ds(j * _CH, _CH)], sem))
      copies.append(pltpu.async_copy(
          ib_hbm.at[idx_i.at[j]], b_i.at[pl.ds(j * _CH, _CH)], sem))
    for c in copies:
      c.wait()

    lane = lax.iota(jnp.int32, _L)

    def group_body(g, _):
      r0 = g * _L
      row_ids = r0 + lane
      acc = b_u[pl.ds(r0, _L)] + b_i[pl.ds(r0, _L)]
      for f in range(_F):
        f_ids = jnp.full((_L,), f, jnp.int32)
        u = plsc.load_gather(rows_u, [row_ids, f_ids])
        v = plsc.load_gather(rows_i, [row_ids, f_ids])
        acc = acc + u * v
      out_v[pl.ds(r0, _L)] = acc
      return 0

    lax.fori_loop(0, n_groups, group_body, 0)
    pltpu.sync_copy(out_v, out_hbm.at[pl.ds(base, b_per_w)])

  return mf_kernel


def kernel(user, item, user_factors, item_factors, user_bias, item_bias):
  mf = _build(user.shape[0])
  return mf(user, item, user_factors, item_factors,
            user_bias.reshape(-1), item_bias.reshape(-1))


# speedup vs baseline: 2.5859x; 2.5859x over previous
import functools

import jax
import jax.numpy as jnp
from jax import lax
from jax.experimental import pallas as pl
from jax.experimental.pallas import tpu as pltpu
from jax.experimental.pallas import tpu_sc as plsc

_NC = 2
_NS = 16
_NW = _NC * _NS
_L = 16
_F = 32
_CH = 128


@functools.lru_cache(maxsize=None)
def _build(batch: int, n_rows: int):
  assert batch % (_NW * _CH) == 0
  b_per_w = batch // _NW
  n_idx_ch = b_per_w // _CH
  n_groups = b_per_w // _L
  mesh = plsc.VectorSubcoreMesh(core_axis_name="c", subcore_axis_name="s")

  @functools.partial(
      pl.kernel,
      mesh=mesh,
      out_type=jax.ShapeDtypeStruct((batch,), jnp.float32),
      compiler_params=pltpu.CompilerParams(
          needs_layout_passes=False, use_tc_tiling_on_sc=True),
      scratch_types=[
          pltpu.VMEM((n_idx_ch, _CH), jnp.int32),
          pltpu.VMEM((n_idx_ch, _CH), jnp.int32),
          pltpu.VMEM((8, 4, 8, 128), jnp.float32),
          pltpu.VMEM((8, 4, 8, 128), jnp.float32),
          pltpu.VMEM((b_per_w,), jnp.float32),
          pltpu.VMEM((b_per_w,), jnp.float32),
          pltpu.VMEM((b_per_w + _L,), jnp.float32),
          pltpu.SemaphoreType.DMA,
          pltpu.SemaphoreType.DMA,
      ],
  )
  def mf_kernel(user_hbm, item_hbm, uf_hbm, if_hbm, ub_hbm, ib_hbm,
                out_hbm, idx_u, idx_i, st_u, st_i, b_u, b_i, out_v, sem,
                sem_b):
    wid = lax.axis_index("s") * _NC + lax.axis_index("c")
    base = wid * b_per_w

    bias_copies = []
    for c in range(n_idx_ch):
      pltpu.sync_copy(user_hbm.at[pl.ds(base + c * _CH, _CH)], idx_u.at[c])
      pltpu.sync_copy(item_hbm.at[pl.ds(base + c * _CH, _CH)], idx_i.at[c])
      bias_copies.append(pltpu.async_copy(
          ub_hbm.at[idx_u.at[c]], b_u.at[pl.ds(c * _CH, _CH)], sem_b))
      bias_copies.append(pltpu.async_copy(
          ib_hbm.at[idx_i.at[c]], b_i.at[pl.ds(c * _CH, _CH)], sem_b))

    lane = lax.iota(jnp.int32, _L)
    half = lane // 8
    eslot = lane % 8
    lo8 = lane % 8
    hi8 = lane % 8 + 8
    lmask = lane < 8
    xor8 = lane ^ 8

    def pair_body(s, _):
      c = s // 8
      p16 = (s % 8) * _L
      uv16 = idx_u[c, pl.ds(p16, _L)]
      iv16 = idx_i[c, pl.ds(p16, _L)]

      def half_chunk(off, dup):
        cps = []
        for l in range(8):
          us = pl.multiple_of((uv16[off + l] >> 7) * 128, 128)
          is_ = pl.multiple_of((iv16[off + l] >> 7) * 128, 128)
          cps.append(pltpu.async_copy(
              uf_hbm.at[:, :, pl.ds(us, 128)], st_u.at[l], sem))
          cps.append(pltpu.async_copy(
              if_hbm.at[:, :, pl.ds(is_, 128)], st_i.at[l], sem))
          if l >= 2:
            cps[2 * (l - 2)].wait()
            cps[2 * (l - 2) + 1].wait()
        for cp in cps[12:]:
          cp.wait()
        ucol = uv16.at[dup].get(mode="promise_in_bounds") & 127
        icol = iv16.at[dup].get(mode="promise_in_bounds") & 127
        acc = jnp.zeros((_L,), jnp.float32)
        for k in range(_F // 2):
          kk = k + half * (_F // 2)
          kft = kk // 8
          kf8 = kk % 8
          acc = acc + (plsc.load_gather(st_u, [eslot, kft, kf8, ucol])
                       * plsc.load_gather(st_i, [eslot, kft, kf8, icol]))
        folded = acc + acc.at[xor8].get(mode="promise_in_bounds")
        plsc.store_compressed(
            out_v.at[pl.ds(s * 2 * 8 + off, _L)], folded, mask=lmask)

      half_chunk(0, lo8)
      half_chunk(8, hi8)
      return 0

    lax.fori_loop(0, b_per_w // _L, pair_body, 0)

    for cp in bias_copies:
      cp.wait()

    def bias_body(g, _):
      e0 = g * _L
      out_v[pl.ds(e0, _L)] = (out_v[pl.ds(e0, _L)]
                              + b_u[pl.ds(e0, _L)] + b_i[pl.ds(e0, _L)])
      return 0

    lax.fori_loop(0, n_groups, bias_body, 0)
    pltpu.sync_copy(out_v.at[pl.ds(0, b_per_w)],
                    out_hbm.at[pl.ds(base, b_per_w)])

  return mf_kernel


def kernel(user, item, user_factors, item_factors, user_bias, item_bias):
  n_rows = user_factors.shape[0]
  mf = _build(user.shape[0], n_rows)
  uf3 = user_factors.T.reshape(4, 8, n_rows)
  if3 = item_factors.T.reshape(4, 8, n_rows)
  return mf(user, item, uf3, if3,
            user_bias.reshape(-1), item_bias.reshape(-1))


# speedup vs baseline: 2.7740x; 1.0727x over previous
import functools

import jax
import jax.numpy as jnp
from jax import lax
from jax.experimental import pallas as pl
from jax.experimental.pallas import tpu as pltpu
from jax.experimental.pallas import tpu_sc as plsc

_NC = 2
_NS = 16
_NW = _NC * _NS
_L = 16
_F = 32
_CH = 128


@functools.lru_cache(maxsize=None)
def _build(batch: int, n_rows: int):
  assert batch % (_NW * _CH) == 0
  b_per_w = batch // _NW
  n_idx_ch = b_per_w // _CH
  n_groups = b_per_w // _L
  mesh = plsc.VectorSubcoreMesh(core_axis_name="c", subcore_axis_name="s")

  @functools.partial(
      pl.kernel,
      mesh=mesh,
      out_type=jax.ShapeDtypeStruct((batch,), jnp.float32),
      compiler_params=pltpu.CompilerParams(
          needs_layout_passes=False, use_tc_tiling_on_sc=True),
      scratch_types=[
          pltpu.VMEM((n_idx_ch, _CH), jnp.int32),
          pltpu.VMEM((n_idx_ch, _CH), jnp.int32),
          pltpu.VMEM((8, 4, 8, 128), jnp.float32),
          pltpu.VMEM((8, 4, 8, 128), jnp.float32),
          pltpu.VMEM((b_per_w,), jnp.float32),
          pltpu.VMEM((b_per_w,), jnp.float32),
          pltpu.VMEM((b_per_w + _L,), jnp.float32),
          pltpu.SemaphoreType.DMA,
          pltpu.SemaphoreType.DMA,
      ],
  )
  def mf_kernel(user_hbm, item_hbm, uf_hbm, if_hbm, ub_hbm, ib_hbm,
                out_hbm, idx_u, idx_i, st_u, st_i, b_u, b_i, out_v, sem,
                sem_b):
    wid = lax.axis_index("s") * _NC + lax.axis_index("c")
    base = wid * b_per_w

    bias_copies = []
    for c in range(n_idx_ch):
      pltpu.sync_copy(user_hbm.at[pl.ds(base + c * _CH, _CH)], idx_u.at[c])
      pltpu.sync_copy(item_hbm.at[pl.ds(base + c * _CH, _CH)], idx_i.at[c])
      bias_copies.append(pltpu.async_copy(
          ub_hbm.at[idx_u.at[c]], b_u.at[pl.ds(c * _CH, _CH)], sem_b))
      bias_copies.append(pltpu.async_copy(
          ib_hbm.at[idx_i.at[c]], b_i.at[pl.ds(c * _CH, _CH)], sem_b))

    lane = lax.iota(jnp.int32, _L)
    half = lane // 8
    eslot = lane % 8
    lo8 = lane % 8
    hi8 = lane % 8 + 8
    lmask = lane < 8
    xor8 = lane ^ 8

    def pair_body(s, _):
      c = s // 8
      p16 = (s % 8) * _L
      uv16 = idx_u[c, pl.ds(p16, _L)]
      iv16 = idx_i[c, pl.ds(p16, _L)]

      def half_chunk(off, dup):
        cps = []
        for l in range(8):
          us = pl.multiple_of((uv16[off + l] >> 7) * 128, 128)
          is_ = pl.multiple_of((iv16[off + l] >> 7) * 128, 128)
          cps.append(pltpu.async_copy(
              uf_hbm.at[:, :, pl.ds(us, 128)], st_u.at[l], sem))
          cps.append(pltpu.async_copy(
              if_hbm.at[:, :, pl.ds(is_, 128)], st_i.at[l], sem))
          if l >= 5:
            cps[2 * (l - 5)].wait()
            cps[2 * (l - 5) + 1].wait()
        for cp in cps[6:]:
          cp.wait()
        ucol = uv16.at[dup].get(mode="promise_in_bounds") & 127
        icol = iv16.at[dup].get(mode="promise_in_bounds") & 127
        acc = jnp.zeros((_L,), jnp.float32)
        for k in range(_F // 2):
          kk = k + half * (_F // 2)
          kft = kk // 8
          kf8 = kk % 8
          acc = acc + (plsc.load_gather(st_u, [eslot, kft, kf8, ucol])
                       * plsc.load_gather(st_i, [eslot, kft, kf8, icol]))
        folded = acc + acc.at[xor8].get(mode="promise_in_bounds")
        plsc.store_compressed(
            out_v.at[pl.ds(s * 2 * 8 + off, _L)], folded, mask=lmask)

      half_chunk(0, lo8)
      half_chunk(8, hi8)
      return 0

    lax.fori_loop(0, b_per_w // _L, pair_body, 0)

    for cp in bias_copies:
      cp.wait()

    def bias_body(g, _):
      e0 = g * _L
      out_v[pl.ds(e0, _L)] = (out_v[pl.ds(e0, _L)]
                              + b_u[pl.ds(e0, _L)] + b_i[pl.ds(e0, _L)])
      return 0

    lax.fori_loop(0, n_groups, bias_body, 0)
    pltpu.sync_copy(out_v.at[pl.ds(0, b_per_w)],
                    out_hbm.at[pl.ds(base, b_per_w)])

  return mf_kernel


def kernel(user, item, user_factors, item_factors, user_bias, item_bias):
  n_rows = user_factors.shape[0]
  mf = _build(user.shape[0], n_rows)
  uf3 = user_factors.T.reshape(4, 8, n_rows)
  if3 = item_factors.T.reshape(4, 8, n_rows)
  return mf(user, item, uf3, if3,
            user_bias.reshape(-1), item_bias.reshape(-1))
